# baseline (speedup 1.0000x reference)
import jax
import jax.numpy as jnp
from jax.experimental import pallas as pl


def _superpose_block(w_ref, k_ref, o_ref):
    w = w_ref[...]
    m = jnp.max(w, axis=-1, keepdims=True)
    e = jnp.exp(w - m)
    probs = e / jnp.sum(e, axis=-1, keepdims=True)
    o_ref[...] = jax.lax.dot_general(
        probs, k_ref[...],
        dimension_numbers=(((1,), (0,)), ((), ())),
        preferred_element_type=jnp.float32)


def kernel(weights, kernel):
    E, H, W = kernel.shape
    B = weights.shape[0]
    N = H * W
    kflat = kernel.reshape(E, N)

    BLK = 64 * 1024
    grid = (N // BLK,)
    out = pl.pallas_call(
        _superpose_block,
        grid=grid,
        in_specs=[
            pl.BlockSpec((B, E), lambda i: (0, 0)),
            pl.BlockSpec((E, BLK), lambda i: (0, i)),
        ],
        out_specs=pl.BlockSpec((B, BLK), lambda i: (0, i)),
        out_shape=jax.ShapeDtypeStruct((B, N), jnp.float32),
    )(weights, kflat)
    return out.reshape(B, H, W)


# speedup vs baseline: 3.0227x; 3.0227x over previous
import jax
import jax.numpy as jnp
from jax.experimental import pallas as pl


def _superpose_block(w_ref, k_ref, o_ref):
    w = w_ref[...]
    m = jnp.max(w, axis=-1, keepdims=True)
    e = jnp.exp(w - m)
    probs = e / jnp.sum(e, axis=-1, keepdims=True)
    o_ref[...] = jax.lax.dot_general(
        probs, k_ref[...],
        dimension_numbers=(((1,), (0,)), ((), ())),
        preferred_element_type=jnp.float32)


def kernel(weights, kernel):
    E, H, W = kernel.shape
    B = weights.shape[0]

    BI = 64
    grid = (H // BI,)
    out = pl.pallas_call(
        _superpose_block,
        grid=grid,
        in_specs=[
            pl.BlockSpec((B, E), lambda i: (0, 0)),
            pl.BlockSpec((E, BI, W), lambda i: (0, i, 0)),
        ],
        out_specs=pl.BlockSpec((B, BI, W), lambda i: (0, i, 0)),
        out_shape=jax.ShapeDtypeStruct((B, H, W), jnp.float32),
    )(weights, kernel)
    return out
